# baseline (speedup 1.0000x reference)
import jax
import jax.numpy as jnp
from jax.experimental import pallas as pl
from jax.experimental.pallas import tpu as pltpu

_BM = 80
_NS = 5


def _main_kernel(*refs):
    adj_refs = refs[:_NS]
    (xb_ref, xrow_ref, w1t_ref, w2t_ref, g_ref, b_ref,
     out_ref, sum_ref, sq_ref) = refs[_NS:]
    i = pl.program_id(0)
    nsteps = pl.num_programs(0)

    @pl.when(i == 0)
    def _init():
        sum_ref[...] = jnp.zeros_like(sum_ref)
        sq_ref[...] = jnp.zeros_like(sq_ref)

    xb = xb_ref[...]
    w1t = w1t_ref[...]
    w2t = w2t_ref[...]
    row0 = i * _NS * _BM
    s = jnp.zeros_like(sum_ref)
    q = jnp.zeros_like(sq_ref)
    for j, a_ref in enumerate(adj_refs):
        a = a_ref[...].astype(jnp.bfloat16)
        agg = jnp.dot(a, xb, preferred_element_type=jnp.float32)
        proj = jnp.dot(xrow_ref[j * _BM:(j + 1) * _BM], w1t,
                       preferred_element_type=jnp.float32)
        proj += jnp.dot(agg, w2t, preferred_element_type=jnp.float32)
        out_ref[pl.ds(row0 + j * _BM, _BM)] = proj
        s += jnp.sum(proj, axis=0, keepdims=True)
        q += jnp.sum(proj * proj, axis=0, keepdims=True)
    sum_ref[...] += s
    sq_ref[...] += q

    @pl.when(i == nsteps - 1)
    def _finalize():
        inv_n = 1.0 / out_ref.shape[0]
        mean = sum_ref[...] * inv_n
        var = sq_ref[...] * inv_n - mean * mean
        scale = g_ref[...] * jax.lax.rsqrt(var + 1e-5)
        shift = b_ref[...] - mean * scale
        out_ref[...] = out_ref[...] * scale + shift


def kernel(x, adj, W, gamma, beta):
    n, d_in = x.shape
    d_out = W.shape[0]
    w1t = W[:, :d_in].T
    w2t = W[:, d_in:].T
    xb = x.astype(jnp.bfloat16)

    def adj_spec(j):
        return pl.BlockSpec((_BM, n), lambda i, j=j: (_NS * i + j, 0))

    return pl.pallas_call(
        _main_kernel,
        grid=(n // (_NS * _BM),),
        in_specs=[adj_spec(j) for j in range(_NS)] + [
            pl.BlockSpec((n, d_in), lambda i: (0, 0)),
            pl.BlockSpec((_NS * _BM, d_in), lambda i: (i, 0)),
            pl.BlockSpec((d_in, d_out), lambda i: (0, 0)),
            pl.BlockSpec((d_in, d_out), lambda i: (0, 0)),
            pl.BlockSpec((1, d_out), lambda i: (0, 0)),
            pl.BlockSpec((1, d_out), lambda i: (0, 0)),
        ],
        out_specs=pl.BlockSpec((n, d_out), lambda i: (0, 0)),
        out_shape=jax.ShapeDtypeStruct((n, d_out), jnp.float32),
        scratch_shapes=[
            pltpu.VMEM((1, d_out), jnp.float32),
            pltpu.VMEM((1, d_out), jnp.float32),
        ],
    )(*([adj] * _NS), xb, x, w1t, w2t,
      gamma.reshape(1, d_out), beta.reshape(1, d_out))
